# baseline (speedup 1.0000x reference)
import jax
import jax.numpy as jnp
from jax import lax
from jax.experimental import pallas as pl
from jax.experimental.pallas import tpu as pltpu
from jax.experimental.pallas import tpu_sc as plsc

_TEMPERATURE = 0.07
_BASE_TEMPERATURE = 0.07

_B = 128
_M = 16384
_D = 2048
_BLK = 512
_NBLK = _M // _BLK

_INT_MAX = 2147483647


def _stats_body(logits_ref, cid_ref, tid_ref, cam_ref, trk_ref,
                loss_ref, hidx_ref,
                m_scr, s_scr, ps_scr, np_scr, hmin_scr, hidx_scr):
    j = pl.program_id(0)

    @pl.when(j == 0)
    def _init():
        m_scr[...] = jnp.full(m_scr.shape, -jnp.inf, m_scr.dtype)
        s_scr[...] = jnp.zeros(s_scr.shape, s_scr.dtype)
        ps_scr[...] = jnp.zeros(ps_scr.shape, ps_scr.dtype)
        np_scr[...] = jnp.zeros(np_scr.shape, np_scr.dtype)
        hmin_scr[...] = jnp.full(hmin_scr.shape, jnp.inf, hmin_scr.dtype)
        hidx_scr[...] = jnp.zeros(hidx_scr.shape, hidx_scr.dtype)

    logits = logits_ref[...]
    cid = cid_ref[...]
    cam = cid == cam_ref[...]
    keyrow = (cid << 16) | tid_ref[...]
    keycol = (cam_ref[...] << 16) | trk_ref[...]
    pos = keyrow == keycol

    a = logits * jnp.float32(1.0 / _TEMPERATURE)

    blk_max = jnp.max(jnp.where(cam, a, -jnp.inf), axis=1, keepdims=True)
    m_old = m_scr[...]
    m_new = jnp.maximum(m_old, blk_max)
    scale = jnp.where(m_old == m_new, jnp.float32(1.0), jnp.exp(m_old - m_new))
    blk_sum = jnp.sum(jnp.where(cam, jnp.exp(a - m_new), 0.0),
                      axis=1, keepdims=True)
    s_scr[...] = s_scr[...] * scale + blk_sum
    m_scr[...] = m_new

    ps_scr[...] = ps_scr[...] + jnp.sum(jnp.where(pos, a, 0.0),
                                        axis=1, keepdims=True)
    np_scr[...] = np_scr[...] + jnp.sum(jnp.where(pos, 1.0, 0.0),
                                        axis=1, keepdims=True)

    v = jnp.where(pos, a, jnp.inf)
    idx = lax.broadcasted_iota(jnp.int32, v.shape, 1) + j * _BLK
    w = _BLK // 2
    while w >= 128:
        v1, v2 = v[:, :w], v[:, w:]
        i1, i2 = idx[:, :w], idx[:, w:]
        lt = v2 < v1
        eq = v2 == v1
        v = jnp.minimum(v1, v2)
        idx = jnp.where(lt, i2, jnp.where(eq, jnp.minimum(i1, i2), i1))
        w //= 2
    blk_min = jnp.min(v, axis=1, keepdims=True)
    blk_arg = jnp.min(jnp.where(v == blk_min, idx, jnp.int32(_INT_MAX)),
                      axis=1, keepdims=True)
    better = blk_min < hmin_scr[...]
    tie = jnp.logical_and(blk_min == hmin_scr[...], blk_arg < hidx_scr[...])
    upd = jnp.logical_or(better, tie)
    hidx_scr[...] = jnp.where(upd, blk_arg, hidx_scr[...])
    hmin_scr[...] = jnp.where(better, blk_min, hmin_scr[...])

    @pl.when(j == _NBLK - 1)
    def _fin():
        mean_lp = ps_scr[...] / np_scr[...] - (m_scr[...] + jnp.log(s_scr[...]))
        loss_i = -(_TEMPERATURE / _BASE_TEMPERATURE) * mean_lp
        loss_ref[...] = jnp.sum(loss_i, axis=0, keepdims=True) * jnp.float32(1.0 / _B)
        hidx_ref[...] = hidx_scr[...]


_stats_call = pl.pallas_call(
    _stats_body,
    grid=(_NBLK,),
    in_specs=[
        pl.BlockSpec((_B, _BLK), lambda j: (0, j)),
        pl.BlockSpec((1, _BLK), lambda j: (0, j)),
        pl.BlockSpec((1, _BLK), lambda j: (0, j)),
        pl.BlockSpec((_B, 1), lambda j: (0, 0)),
        pl.BlockSpec((_B, 1), lambda j: (0, 0)),
    ],
    out_specs=[
        pl.BlockSpec((1, 1), lambda j: (0, 0)),
        pl.BlockSpec((_B, 1), lambda j: (0, 0)),
    ],
    out_shape=[
        jax.ShapeDtypeStruct((1, 1), jnp.float32),
        jax.ShapeDtypeStruct((_B, 1), jnp.int32),
    ],
    scratch_shapes=[
        pltpu.VMEM((_B, 1), jnp.float32),
        pltpu.VMEM((_B, 1), jnp.float32),
        pltpu.VMEM((_B, 1), jnp.float32),
        pltpu.VMEM((_B, 1), jnp.float32),
        pltpu.VMEM((_B, 1), jnp.float32),
        pltpu.VMEM((_B, 1), jnp.int32),
    ],
)

_NC = 1
_NS = 16
_NW = _NC * _NS
_RPW = _B // _NW


def _gather_body(mem_hbm, idx_hbm, out_hbm, idx_v, rows_v, sem):
    wid = lax.axis_index("s") * _NC + lax.axis_index("c")
    base = wid * _RPW
    pltpu.sync_copy(idx_hbm.at[pl.ds(base, _RPW)], idx_v)
    pltpu.async_copy(mem_hbm.at[idx_v], rows_v, sem).wait()
    pltpu.sync_copy(rows_v, out_hbm.at[pl.ds(base, _RPW)])


_gather_call = pl.kernel(
    _gather_body,
    out_type=jax.ShapeDtypeStruct((_B, _D), jnp.float32),
    mesh=plsc.VectorSubcoreMesh(core_axis_name="c", subcore_axis_name="s",
                                num_cores=_NC),
    scratch_types=[
        pltpu.VMEM((_RPW,), jnp.int32),
        pltpu.VMEM((_RPW, _D), jnp.float32),
        pltpu.SemaphoreType.DMA,
    ],
)


def kernel(mem, logits, mem_CID, mem_TID, camids, trackids):
    loss2, hidx2 = _stats_call(
        logits,
        mem_CID.reshape(1, _M),
        mem_TID.reshape(1, _M),
        camids.reshape(_B, 1),
        trackids.reshape(_B, 1),
    )
    hard_pos = _gather_call(mem, hidx2.reshape(_B))
    return loss2[0, 0], hard_pos


# speedup vs baseline: 1.4319x; 1.4319x over previous
import jax
import jax.numpy as jnp
from jax import lax
from jax.experimental import pallas as pl
from jax.experimental.pallas import tpu as pltpu
from jax.experimental.pallas import tpu_sc as plsc

_TEMPERATURE = 0.07
_BASE_TEMPERATURE = 0.07

_B = 128
_M = 16384
_D = 2048
_BLK = 4096
_NBLK = _M // _BLK

_INT_MAX = 2147483647


def _stats_body(logits_ref, cid_ref, tid_ref, cam_ref, trk_ref,
                loss_ref, hidx_ref,
                m_scr, s_scr, ps_scr, np_scr, hmin_scr, hidx_scr):
    j = pl.program_id(0)

    @pl.when(j == 0)
    def _init():
        m_scr[...] = jnp.full(m_scr.shape, -jnp.inf, m_scr.dtype)
        s_scr[...] = jnp.zeros(s_scr.shape, s_scr.dtype)
        ps_scr[...] = jnp.zeros(ps_scr.shape, ps_scr.dtype)
        np_scr[...] = jnp.zeros(np_scr.shape, np_scr.dtype)
        hmin_scr[...] = jnp.full(hmin_scr.shape, jnp.inf, hmin_scr.dtype)
        hidx_scr[...] = jnp.zeros(hidx_scr.shape, hidx_scr.dtype)

    logits = logits_ref[...]
    cid = cid_ref[...]
    cam = cid == cam_ref[...]
    keyrow = (cid << 16) | tid_ref[...]
    keycol = (cam_ref[...] << 16) | trk_ref[...]
    pos = keyrow == keycol

    a = logits * jnp.float32(1.0 / _TEMPERATURE)

    blk_max = jnp.max(jnp.where(cam, a, -jnp.inf), axis=1, keepdims=True)
    m_old = m_scr[...]
    m_new = jnp.maximum(m_old, blk_max)
    scale = jnp.where(m_old == m_new, jnp.float32(1.0), jnp.exp(m_old - m_new))
    blk_sum = jnp.sum(jnp.where(cam, jnp.exp(a - m_new), 0.0),
                      axis=1, keepdims=True)
    s_scr[...] = s_scr[...] * scale + blk_sum
    m_scr[...] = m_new

    ps_scr[...] = ps_scr[...] + jnp.sum(jnp.where(pos, a, 0.0),
                                        axis=1, keepdims=True)
    np_scr[...] = np_scr[...] + jnp.sum(jnp.where(pos, 1.0, 0.0),
                                        axis=1, keepdims=True)

    v = jnp.where(pos, a, jnp.inf)
    idx = lax.broadcasted_iota(jnp.int32, v.shape, 1) + j * _BLK
    w = _BLK // 2
    while w >= 128:
        v1, v2 = v[:, :w], v[:, w:]
        i1, i2 = idx[:, :w], idx[:, w:]
        lt = v2 < v1
        eq = v2 == v1
        v = jnp.minimum(v1, v2)
        idx = jnp.where(lt, i2, jnp.where(eq, jnp.minimum(i1, i2), i1))
        w //= 2
    blk_min = jnp.min(v, axis=1, keepdims=True)
    blk_arg = jnp.min(jnp.where(v == blk_min, idx, jnp.int32(_INT_MAX)),
                      axis=1, keepdims=True)
    better = blk_min < hmin_scr[...]
    tie = jnp.logical_and(blk_min == hmin_scr[...], blk_arg < hidx_scr[...])
    upd = jnp.logical_or(better, tie)
    hidx_scr[...] = jnp.where(upd, blk_arg, hidx_scr[...])
    hmin_scr[...] = jnp.where(better, blk_min, hmin_scr[...])

    @pl.when(j == _NBLK - 1)
    def _fin():
        mean_lp = ps_scr[...] / np_scr[...] - (m_scr[...] + jnp.log(s_scr[...]))
        loss_i = -(_TEMPERATURE / _BASE_TEMPERATURE) * mean_lp
        loss_ref[...] = jnp.sum(loss_i, axis=0, keepdims=True) * jnp.float32(1.0 / _B)
        hidx_ref[...] = hidx_scr[...]


_stats_call = pl.pallas_call(
    _stats_body,
    grid=(_NBLK,),
    in_specs=[
        pl.BlockSpec((_B, _BLK), lambda j: (0, j)),
        pl.BlockSpec((1, _BLK), lambda j: (0, j)),
        pl.BlockSpec((1, _BLK), lambda j: (0, j)),
        pl.BlockSpec((_B, 1), lambda j: (0, 0)),
        pl.BlockSpec((_B, 1), lambda j: (0, 0)),
    ],
    out_specs=[
        pl.BlockSpec((1, 1), lambda j: (0, 0)),
        pl.BlockSpec((_B, 1), lambda j: (0, 0)),
    ],
    out_shape=[
        jax.ShapeDtypeStruct((1, 1), jnp.float32),
        jax.ShapeDtypeStruct((_B, 1), jnp.int32),
    ],
    scratch_shapes=[
        pltpu.VMEM((_B, 1), jnp.float32),
        pltpu.VMEM((_B, 1), jnp.float32),
        pltpu.VMEM((_B, 1), jnp.float32),
        pltpu.VMEM((_B, 1), jnp.float32),
        pltpu.VMEM((_B, 1), jnp.float32),
        pltpu.VMEM((_B, 1), jnp.int32),
    ],
)

_NC = 1
_NS = 16
_NW = _NC * _NS
_RPW = _B // _NW


def _gather_body(mem_hbm, idx_hbm, out_hbm, idx_v, rows_v, sem):
    wid = lax.axis_index("s") * _NC + lax.axis_index("c")
    base = wid * _RPW
    pltpu.sync_copy(idx_hbm.at[pl.ds(base, _RPW)], idx_v)
    pltpu.async_copy(mem_hbm.at[idx_v], rows_v, sem).wait()
    pltpu.sync_copy(rows_v, out_hbm.at[pl.ds(base, _RPW)])


_gather_call = pl.kernel(
    _gather_body,
    out_type=jax.ShapeDtypeStruct((_B, _D), jnp.float32),
    mesh=plsc.VectorSubcoreMesh(core_axis_name="c", subcore_axis_name="s",
                                num_cores=_NC),
    scratch_types=[
        pltpu.VMEM((_RPW,), jnp.int32),
        pltpu.VMEM((_RPW, _D), jnp.float32),
        pltpu.SemaphoreType.DMA,
    ],
)


def kernel(mem, logits, mem_CID, mem_TID, camids, trackids):
    loss2, hidx2 = _stats_call(
        logits,
        mem_CID.reshape(1, _M),
        mem_TID.reshape(1, _M),
        camids.reshape(_B, 1),
        trackids.reshape(_B, 1),
    )
    hard_pos = _gather_call(mem, hidx2.reshape(_B))
    return loss2[0, 0], hard_pos
